# baseline (speedup 1.0000x reference)
import jax
import jax.numpy as jnp
from jax.experimental import pallas as pl
from jax.experimental.pallas import tpu as pltpu


def _se_kernel(x_ref, w1_ref, w2_ref, bias_ref, o_ref):
    nb = x_ref.shape[0]
    cr = w1_ref.shape[0]
    inv_hw = jnp.float32(1.0 / x_ref.shape[-1])
    b1 = bias_ref[:cr, 0:1]
    alpha = bias_ref[:cr, 1:2]
    b2 = bias_ref[:, 2:3]

    for n in range(nb):
        pooled = jnp.sum(x_ref[n], axis=-1, keepdims=True) * inv_hw
        h = jnp.dot(w1_ref[...], pooled,
                    preferred_element_type=jnp.float32) + b1
        h = jnp.where(h >= 0, h, alpha * h)
        y = jnp.dot(w2_ref[...], h,
                    preferred_element_type=jnp.float32) + b2
        gate = jax.nn.sigmoid(y)
        o_ref[n] = x_ref[n] * gate


def kernel(x_nchw, w1, b1, alpha, w2, b2):
    N, C, H, W = x_nchw.shape
    HW = H * W
    Cr = w1.shape[0]

    x3 = x_nchw.reshape(N, C, HW)
    itemsize = jnp.dtype(x3.dtype).itemsize
    nb = 4

    bias_pack = jnp.zeros((C, 3), jnp.float32)
    bias_pack = bias_pack.at[:Cr, 0].set(b1).at[:Cr, 1].set(alpha)
    bias_pack = bias_pack.at[:, 2].set(b2)

    param_bytes = int((w1.size + w2.size + b1.size + b2.size + alpha.size) * 4)
    cost = pl.CostEstimate(
        flops=int(2 * N * C * HW + 4 * N * C * Cr),
        transcendentals=int(N * C),
        bytes_accessed=int(2 * N * C * HW * itemsize + param_bytes),
    )

    out3 = pl.pallas_call(
        _se_kernel,
        out_shape=jax.ShapeDtypeStruct((N, C, HW), x3.dtype),
        grid_spec=pltpu.PrefetchScalarGridSpec(
            num_scalar_prefetch=0,
            grid=(N // nb,),
            in_specs=[
                pl.BlockSpec((nb, C, HW), lambda i: (i, 0, 0)),
                pl.BlockSpec((Cr, C), lambda i: (0, 0)),
                pl.BlockSpec((C, Cr), lambda i: (0, 0)),
                pl.BlockSpec((C, 3), lambda i: (0, 0)),
            ],
            out_specs=pl.BlockSpec((nb, C, HW), lambda i: (i, 0, 0)),
        ),
        compiler_params=pltpu.CompilerParams(
            dimension_semantics=("parallel",),
            vmem_limit_bytes=48 * 1024 * 1024,
        ),
        cost_estimate=cost,
    )(x3, w1, w2, bias_pack)

    return out3.reshape(N, C, H, W)


# speedup vs baseline: 1.0150x; 1.0150x over previous
import jax
import jax.numpy as jnp
from jax.experimental import pallas as pl
from jax.experimental.pallas import tpu as pltpu


def _se_kernel(x_ref, w1_ref, b1_ref, alpha_ref, w2_ref, b2_ref, o_ref):
    nb = x_ref.shape[0]
    inv_hw = jnp.float32(1.0 / x_ref.shape[-1])

    for n in range(nb):
        pooled = jnp.sum(x_ref[n], axis=-1, keepdims=True) * inv_hw
        h = jnp.dot(w1_ref[...], pooled,
                    preferred_element_type=jnp.float32) + b1_ref[...]
        h = jnp.where(h >= 0, h, alpha_ref[...] * h)
        y = jnp.dot(w2_ref[...], h,
                    preferred_element_type=jnp.float32) + b2_ref[...]
        gate = jax.nn.sigmoid(y)
        o_ref[n] = x_ref[n] * gate


def kernel(x_nchw, w1, b1, alpha, w2, b2):
    N, C, H, W = x_nchw.shape
    HW = H * W
    Cr = w1.shape[0]

    x3 = x_nchw.reshape(N, C, HW)
    itemsize = jnp.dtype(x3.dtype).itemsize
    nb = 4

    param_bytes = int((w1.size + w2.size + b1.size + b2.size + alpha.size) * 4)
    cost = pl.CostEstimate(
        flops=int(2 * N * C * HW + 4 * N * C * Cr),
        transcendentals=int(N * C),
        bytes_accessed=int(2 * N * C * HW * itemsize + param_bytes),
    )

    out3 = pl.pallas_call(
        _se_kernel,
        out_shape=jax.ShapeDtypeStruct((N, C, HW), x3.dtype),
        grid_spec=pltpu.PrefetchScalarGridSpec(
            num_scalar_prefetch=0,
            grid=(N // nb,),
            in_specs=[
                pl.BlockSpec((nb, C, HW), lambda i: (i, 0, 0)),
                pl.BlockSpec((Cr, C), lambda i: (0, 0)),
                pl.BlockSpec((Cr, 1), lambda i: (0, 0)),
                pl.BlockSpec((Cr, 1), lambda i: (0, 0)),
                pl.BlockSpec((C, Cr), lambda i: (0, 0)),
                pl.BlockSpec((C, 1), lambda i: (0, 0)),
            ],
            out_specs=pl.BlockSpec((nb, C, HW), lambda i: (i, 0, 0)),
        ),
        compiler_params=pltpu.CompilerParams(
            dimension_semantics=("parallel",),
            vmem_limit_bytes=48 * 1024 * 1024,
        ),
        cost_estimate=cost,
    )(x3, w1, b1.reshape(Cr, 1), alpha.reshape(Cr, 1), w2, b2.reshape(C, 1))

    return out3.reshape(N, C, H, W)
